# Initial kernel scaffold; baseline (speedup 1.0000x reference)
import jax
import jax.numpy as jnp
from jax.experimental import pallas as pl


def kernel(sxl, sxr, idx_l, idx_r):
    raise NotImplementedError("write your pallas kernel here")


# speedup vs baseline: 2.3331x; 2.3331x over previous
import functools

import jax
import jax.numpy as jnp
from jax import lax
from jax.experimental import pallas as pl
from jax.experimental.pallas import tpu as pltpu
from jax.experimental.pallas import tpu_sc as plsc

_NUM_CORES = 2
_NUM_SUBCORES = 16
_LANES = 16


def _tc_corr_matmul(xl3, xr3, t_len, t_tile):
    B, Rl, _ = xl3.shape
    Rr = xr3.shape[1]
    nt = t_len // t_tile

    def body(xl_ref, xr_ref, c_ref):
        t = pl.program_id(1)

        @pl.when(t == 0)
        def _init():
            c_ref[...] = jnp.zeros_like(c_ref)

        a = xl_ref[0]
        b = xr_ref[0]
        c_ref[0] += lax.dot_general(
            a, b, (((1,), (1,)), ((), ())), preferred_element_type=jnp.float32
        )

    return pl.pallas_call(
        body,
        grid=(B, nt),
        in_specs=[
            pl.BlockSpec((1, Rl, t_tile), lambda b, t: (b, 0, t)),
            pl.BlockSpec((1, Rr, t_tile), lambda b, t: (b, 0, t)),
        ],
        out_specs=pl.BlockSpec((1, Rl, Rr), lambda b, t: (b, 0, 0)),
        out_shape=jax.ShapeDtypeStruct((B, Rl, Rr), jnp.float32),
        compiler_params=pltpu.CompilerParams(
            dimension_semantics=("parallel", "arbitrary")
        ),
    )(xl3, xr3)


def _sc_gather(c4, idx_l, idx_r, B, Nl, Nr, J, K, inv_t):
    nw = _NUM_CORES * _NUM_SUBCORES
    n_out = B * Nl * Nr * K
    pairs_per_w = (B * Nl * Nr) // nw
    per_w = pairs_per_w * K
    mesh = plsc.VectorSubcoreMesh(core_axis_name="c", subcore_axis_name="s")

    @functools.partial(
        pl.kernel,
        out_type=jax.ShapeDtypeStruct((n_out,), jnp.float32),
        mesh=mesh,
        scratch_types=[
            pltpu.VMEM((J * Nr * J,), jnp.float32),
            pltpu.VMEM((K,), jnp.int32),
            pltpu.VMEM((K,), jnp.int32),
            pltpu.VMEM((per_w,), jnp.float32),
        ],
        compiler_params=pltpu.CompilerParams(needs_layout_passes=False),
    )
    def k(c4_hbm, il_hbm, ir_hbm, y_hbm, cb_v, il_v, ir_v, out_v):
        wid = lax.axis_index("s") * _NUM_CORES + lax.axis_index("c")
        q0 = wid * pairs_per_w
        row_blk = q0 // Nr
        nr0 = q0 - row_blk * Nr
        pltpu.sync_copy(c4_hbm.at[row_blk], cb_v)
        pltpu.sync_copy(il_hbm, il_v)
        pltpu.sync_copy(ir_hbm, ir_v)
        for p_local in range(pairs_per_w):
            col_base = (nr0 + p_local) * J
            for kv in range(K // _LANES):
                il = il_v[pl.ds(kv * _LANES, _LANES)]
                ir = ir_v[pl.ds(kv * _LANES, _LANES)]
                flat = il * (Nr * J) + (col_base + ir)
                vals = plsc.load_gather(cb_v, [flat])
                out_v[pl.ds(p_local * K + kv * _LANES, _LANES)] = vals * inv_t
        pltpu.sync_copy(out_v, y_hbm.at[pl.ds(wid * per_w, per_w)])

    return k(c4, idx_l, idx_r)


def kernel(sxl, sxr, idx_l, idx_r):
    B, Nl, J, A, T = sxl.shape
    Nr = sxr.shape[1]
    K = idx_l.shape[0]
    xl3 = sxl.reshape(B, Nl * J, A * T)
    xr3 = sxr.reshape(B, Nr * J, A * T)
    c = _tc_corr_matmul(xl3, xr3, T, 1024)
    c4 = c.reshape(B * Nl, J * Nr * J)
    y = _sc_gather(c4, idx_l, idx_r, B, Nl, Nr, J, K, 1.0 / T)
    return y.reshape(B, Nl * Nr, K, 1)


# speedup vs baseline: 2.5963x; 1.1128x over previous
import functools

import jax
import jax.numpy as jnp
from jax import lax
from jax.experimental import pallas as pl
from jax.experimental.pallas import tpu as pltpu
from jax.experimental.pallas import tpu_sc as plsc

_NUM_CORES = 2
_NUM_SUBCORES = 16
_LANES = 16


def _tc_corr_matmul(xl3, xr3, t_len, t_tile):
    B, Rl, _ = xl3.shape
    Rr = xr3.shape[1]
    nt = t_len // t_tile

    def body(xl_ref, xr_ref, c_ref):
        t = pl.program_id(1)

        @pl.when(t == 0)
        def _init():
            c_ref[...] = jnp.zeros_like(c_ref)

        a = xl_ref[0]
        b = xr_ref[0]
        c_ref[0] += lax.dot_general(
            a, b, (((1,), (1,)), ((), ())), preferred_element_type=jnp.float32
        )

    return pl.pallas_call(
        body,
        grid=(B, nt),
        in_specs=[
            pl.BlockSpec((1, Rl, t_tile), lambda b, t: (b, 0, t)),
            pl.BlockSpec((1, Rr, t_tile), lambda b, t: (b, 0, t)),
        ],
        out_specs=pl.BlockSpec((1, Rl, Rr), lambda b, t: (b, 0, 0)),
        out_shape=jax.ShapeDtypeStruct((B, Rl, Rr), jnp.float32),
        compiler_params=pltpu.CompilerParams(
            dimension_semantics=("parallel", "arbitrary")
        ),
    )(xl3, xr3)


def _sc_gather(c4, idx_l, idx_r, B, Nl, Nr, J, K, inv_t):
    nw = _NUM_CORES * _NUM_SUBCORES
    n_out = B * Nl * Nr * K
    pairs_per_w = (B * Nl * Nr) // nw
    per_w = pairs_per_w * K
    mesh = plsc.VectorSubcoreMesh(core_axis_name="c", subcore_axis_name="s")

    @functools.partial(
        pl.kernel,
        out_type=jax.ShapeDtypeStruct((n_out,), jnp.float32),
        mesh=mesh,
        scratch_types=[
            pltpu.VMEM((J * Nr * J,), jnp.float32),
            pltpu.VMEM((K,), jnp.int32),
            pltpu.VMEM((K,), jnp.int32),
            pltpu.VMEM((per_w,), jnp.float32),
        ],
        compiler_params=pltpu.CompilerParams(needs_layout_passes=False),
    )
    def k(c4_hbm, il_hbm, ir_hbm, y_hbm, cb_v, il_v, ir_v, out_v):
        wid = lax.axis_index("s") * _NUM_CORES + lax.axis_index("c")
        q0 = wid * pairs_per_w
        row_blk = q0 // Nr
        nr0 = q0 - row_blk * Nr
        pltpu.sync_copy(c4_hbm.at[row_blk], cb_v)
        pltpu.sync_copy(il_hbm, il_v)
        pltpu.sync_copy(ir_hbm, ir_v)
        for p_local in range(pairs_per_w):
            col_base = (nr0 + p_local) * J
            for kv in range(K // _LANES):
                il = il_v[pl.ds(kv * _LANES, _LANES)]
                ir = ir_v[pl.ds(kv * _LANES, _LANES)]
                flat = il * (Nr * J) + (col_base + ir)
                vals = plsc.load_gather(cb_v, [flat])
                out_v[pl.ds(p_local * K + kv * _LANES, _LANES)] = vals * inv_t
        pltpu.sync_copy(out_v, y_hbm.at[pl.ds(wid * per_w, per_w)])

    return k(c4, idx_l, idx_r)


def kernel(sxl, sxr, idx_l, idx_r):
    B, Nl, J, A, T = sxl.shape
    Nr = sxr.shape[1]
    K = idx_l.shape[0]
    xl3 = sxl.reshape(B, Nl * J, A * T)
    xr3 = sxr.reshape(B, Nr * J, A * T)
    c = _tc_corr_matmul(xl3, xr3, T, 8192)
    c4 = c.reshape(B * Nl, J * Nr * J)
    y = _sc_gather(c4, idx_l, idx_r, B, Nl, Nr, J, K, 1.0 / T)
    return y.reshape(B, Nl * Nr, K, 1)


# speedup vs baseline: 3.3753x; 1.3000x over previous
import functools

import jax
import jax.numpy as jnp
from jax import lax
from jax.experimental import pallas as pl
from jax.experimental.pallas import tpu as pltpu
from jax.experimental.pallas import tpu_sc as plsc

_NUM_CORES = 2
_NUM_SUBCORES = 16
_LANES = 16


def _tc_corr_matmul(xl3, xr3, t_len, t_tile):
    B, Rl, _ = xl3.shape
    Rr = xr3.shape[1]
    nt = t_len // t_tile

    def body(xl_ref, xr_ref, c_ref):
        t = pl.program_id(1)

        @pl.when(t == 0)
        def _init():
            c_ref[...] = jnp.zeros_like(c_ref)

        a = xl_ref[0]
        b = xr_ref[0]
        c_ref[0] += lax.dot_general(
            a, b, (((1,), (1,)), ((), ())), preferred_element_type=jnp.float32
        )

    return pl.pallas_call(
        body,
        grid=(B, nt),
        in_specs=[
            pl.BlockSpec((1, Rl, t_tile), lambda b, t: (b, 0, t)),
            pl.BlockSpec((1, Rr, t_tile), lambda b, t: (b, 0, t)),
        ],
        out_specs=pl.BlockSpec((1, Rl, Rr), lambda b, t: (b, 0, 0)),
        out_shape=jax.ShapeDtypeStruct((B, Rl, Rr), jnp.float32),
        compiler_params=pltpu.CompilerParams(
            dimension_semantics=("parallel", "arbitrary")
        ),
    )(xl3, xr3)


def _sc_gather(c4, idx_l, idx_r, B, Nl, Nr, J, K, inv_t):
    nw = _NUM_CORES * _NUM_SUBCORES
    n_out = B * Nl * Nr * K
    pairs_per_w = (B * Nl * Nr) // nw
    per_w = pairs_per_w * K
    mesh = plsc.VectorSubcoreMesh(core_axis_name="c", subcore_axis_name="s")

    @functools.partial(
        pl.kernel,
        out_type=jax.ShapeDtypeStruct((n_out,), jnp.float32),
        mesh=mesh,
        scratch_types=[
            pltpu.VMEM((J * Nr * J,), jnp.float32),
            pltpu.VMEM((K,), jnp.int32),
            pltpu.VMEM((K,), jnp.int32),
            pltpu.VMEM((per_w,), jnp.float32),
        ],
        compiler_params=pltpu.CompilerParams(needs_layout_passes=False),
    )
    def k(c4_hbm, il_hbm, ir_hbm, y_hbm, cb_v, il_v, ir_v, out_v):
        wid = lax.axis_index("s") * _NUM_CORES + lax.axis_index("c")
        q0 = wid * pairs_per_w
        row_blk = q0 // Nr
        nr0 = q0 - row_blk * Nr
        pltpu.sync_copy(c4_hbm.at[row_blk], cb_v)
        pltpu.sync_copy(il_hbm, il_v)
        pltpu.sync_copy(ir_hbm, ir_v)
        for p_local in range(pairs_per_w):
            col_base = (nr0 + p_local) * J
            for kv in range(K // _LANES):
                il = il_v[pl.ds(kv * _LANES, _LANES)]
                ir = ir_v[pl.ds(kv * _LANES, _LANES)]
                flat = il * (Nr * J) + (col_base + ir)
                vals = plsc.load_gather(cb_v, [flat])
                out_v[pl.ds(p_local * K + kv * _LANES, _LANES)] = vals * inv_t
        pltpu.sync_copy(out_v, y_hbm.at[pl.ds(wid * per_w, per_w)])

    return k(c4, idx_l, idx_r)


def kernel(sxl, sxr, idx_l, idx_r):
    B, Nl, J, A, T = sxl.shape
    Nr = sxr.shape[1]
    K = idx_l.shape[0]
    xl3 = sxl[:, :, :, 0, :].reshape(B, Nl * J, T)
    xr3 = sxr[:, :, :, 0, :].reshape(B, Nr * J, T)
    c = _tc_corr_matmul(xl3, xr3, T, 8192)
    c4 = c.reshape(B * Nl, J * Nr * J)
    y = _sc_gather(c4, idx_l, idx_r, B, Nl, Nr, J, K, 1.0 / T)
    return y.reshape(B, Nl * Nr, K, 1)
